# Initial kernel scaffold; baseline (speedup 1.0000x reference)
import jax
import jax.numpy as jnp
from jax.experimental import pallas as pl


def kernel(x, edge_index, W1, b1, W2, b2):
    raise NotImplementedError("write your pallas kernel here")


# speedup vs baseline: 15.9217x; 15.9217x over previous
import functools

import jax
import jax.numpy as jnp
from jax import lax
from jax.experimental import pallas as pl
from jax.experimental.pallas import tpu as pltpu
from jax.experimental.pallas import tpu_sc as plsc

N_NODES = 10000
D = 128
N_EDGES = 320000

NC = 2
NS = 16
NW = NC * NS
N_PAD = 10240
RPT = N_PAD // NS
EPT = N_EDGES // NW
K = 80
CHUNKS = EPT // K
BLK = 512
GRID = N_PAD // BLK
DEGW = 128

@functools.lru_cache(maxsize=1)
def _sc_kernels():
    mesh = plsc.VectorSubcoreMesh(core_axis_name="c", subcore_axis_name="s")

    @functools.partial(
        pl.kernel,
        mesh=mesh,
        out_type=jax.ShapeDtypeStruct((NC, N_PAD, DEGW), jnp.float32),
        scratch_types=[
            pltpu.VMEM((CHUNKS, K), jnp.int32),
            pltpu.VMEM((K, DEGW), jnp.float32),
            pltpu.VMEM_SHARED((N_PAD, DEGW), jnp.float32),
        ],
    )
    def sc_degree(dst_hbm, ones_hbm, zeros1_hbm, out_hbm, dst_v, ones_v, accum):
        c = lax.axis_index("c")
        s = lax.axis_index("s")
        w = c * NS + s
        pltpu.sync_copy(zeros1_hbm, accum.at[pl.ds(s * RPT, RPT)])
        pltpu.sync_copy(dst_hbm.at[w], dst_v)
        pltpu.sync_copy(ones_hbm, ones_v)
        plsc.subcore_barrier()

        def body(i, carry):
            pltpu.sync_copy(ones_v, accum.at[dst_v.at[i]], add=True)
            return carry

        lax.fori_loop(0, CHUNKS, body, 0)
        plsc.subcore_barrier()
        pltpu.sync_copy(accum.at[pl.ds(s * RPT, RPT)],
                        out_hbm.at[c, pl.ds(s * RPT, RPT)])

    @functools.partial(
        pl.kernel,
        mesh=mesh,
        out_type=jax.ShapeDtypeStruct((NC, N_PAD, D), jnp.float32),
        scratch_types=[
            pltpu.VMEM((CHUNKS, K), jnp.int32),
            pltpu.VMEM((CHUNKS, K), jnp.int32),
            pltpu.VMEM((K, D), jnp.float32),
            pltpu.VMEM_SHARED((N_PAD, D), jnp.float32),
        ],
    )
    def sc_scatter(g_hbm, src_hbm, dst_hbm, zeros_hbm, out_hbm,
                   src_v, dst_v, rows_v, accum):
        c = lax.axis_index("c")
        s = lax.axis_index("s")
        w = c * NS + s
        pltpu.sync_copy(zeros_hbm, accum.at[pl.ds(s * RPT, RPT)])
        pltpu.sync_copy(src_hbm.at[w], src_v)
        pltpu.sync_copy(dst_hbm.at[w], dst_v)
        plsc.subcore_barrier()

        def body(i, carry):
            pltpu.sync_copy(g_hbm.at[src_v.at[i]], rows_v)
            pltpu.sync_copy(rows_v, accum.at[dst_v.at[i]], add=True)
            return carry

        lax.fori_loop(0, CHUNKS, body, 0)
        plsc.subcore_barrier()
        pltpu.sync_copy(accum.at[pl.ds(s * RPT, RPT)],
                        out_hbm.at[c, pl.ds(s * RPT, RPT)])

    return sc_degree, sc_scatter


def _tc_norm_mm_body(x_ref, w_ref, deg_ref, g_ref, d_ref):
    deg = deg_ref[0][:, :1] + deg_ref[1][:, :1] + 1.0
    d = lax.rsqrt(deg)
    d_ref[...] = d
    g_ref[...] = jnp.dot(x_ref[...], w_ref[...],
                         preferred_element_type=jnp.float32) * d


def _tc_mid_body(s_ref, g1_ref, d_ref, b1_ref, w2_ref, g2_ref):
    d = d_ref[...]
    ssum = s_ref[0] + s_ref[1]
    out1 = jnp.maximum(d * (ssum + g1_ref[...]) + b1_ref[...], 0.0)
    g2_ref[...] = jnp.dot(out1, w2_ref[...],
                          preferred_element_type=jnp.float32) * d


def _tc_final_body(s_ref, g2_ref, d_ref, b2_ref, x_ref, o_ref):
    d = d_ref[...]
    ssum = s_ref[0] + s_ref[1]
    o_ref[...] = jnp.maximum(
        d * (ssum + g2_ref[...]) + b2_ref[...] + x_ref[...], 0.0)


_row_spec = pl.BlockSpec((BLK, D), lambda i: (i, 0))
_d_spec = pl.BlockSpec((BLK, 1), lambda i: (i, 0))
_part_spec = pl.BlockSpec((NC, BLK, D), lambda i: (0, i, 0))
_deg_spec = pl.BlockSpec((NC, BLK, DEGW), lambda i: (0, i, 0))
_w_spec = pl.BlockSpec((D, D), lambda i: (0, 0))
_b_spec = pl.BlockSpec((1, D), lambda i: (0, 0))

_tc_norm_mm = pl.pallas_call(
    _tc_norm_mm_body,
    grid=(GRID,),
    in_specs=[_row_spec, _w_spec, _deg_spec],
    out_specs=[_row_spec, _d_spec],
    out_shape=[
        jax.ShapeDtypeStruct((N_NODES, D), jnp.float32),
        jax.ShapeDtypeStruct((N_PAD, 1), jnp.float32),
    ],
)

_tc_mid = pl.pallas_call(
    _tc_mid_body,
    grid=(GRID,),
    in_specs=[_part_spec, _row_spec, _d_spec, _b_spec, _w_spec],
    out_specs=_row_spec,
    out_shape=jax.ShapeDtypeStruct((N_NODES, D), jnp.float32),
)

_tc_final = pl.pallas_call(
    _tc_final_body,
    grid=(GRID,),
    in_specs=[_part_spec, _row_spec, _d_spec, _b_spec, _row_spec],
    out_specs=_row_spec,
    out_shape=jax.ShapeDtypeStruct((N_NODES, D), jnp.float32),
)


@jax.jit
def kernel(x, edge_index, W1, b1, W2, b2):
    src = edge_index[0].astype(jnp.int32).reshape(NW, CHUNKS, K)
    dst = edge_index[1].astype(jnp.int32).reshape(NW, CHUNKS, K)
    ones = jnp.ones((K, DEGW), jnp.float32)
    zeros = jnp.zeros((RPT, D), jnp.float32)
    zeros1 = zeros
    b1r = b1.reshape(1, D)
    b2r = b2.reshape(1, D)

    sc_degree, sc_scatter = _sc_kernels()
    deg_parts = sc_degree(dst, ones, zeros1)
    g1, d = _tc_norm_mm(x, W1, deg_parts)
    s1 = sc_scatter(g1, src, dst, zeros)
    g2 = _tc_mid(s1, g1, d, b1r, W2)
    s2 = sc_scatter(g2, src, dst, zeros)
    return _tc_final(s2, g2, d, b2r, x)


# speedup vs baseline: 19.5140x; 1.2256x over previous
import functools

import jax
import jax.numpy as jnp
from jax import lax
from jax.experimental import pallas as pl
from jax.experimental.pallas import tpu as pltpu
from jax.experimental.pallas import tpu_sc as plsc

N_NODES = 10000
D = 128
N_EDGES = 320000

NC = 2
NS = 16
NW = NC * NS
N_PAD = 10240
RPT = N_PAD // NS
EPT = N_EDGES // NW
K = 80
CHUNKS = EPT // K
BLK = 512
GRID = N_PAD // BLK
DEGW = 128

@functools.lru_cache(maxsize=1)
def _sc_kernels():
    mesh = plsc.VectorSubcoreMesh(core_axis_name="c", subcore_axis_name="s")

    @functools.partial(
        pl.kernel,
        mesh=mesh,
        out_type=jax.ShapeDtypeStruct((NC, N_PAD, DEGW), jnp.float32),
        scratch_types=[
            pltpu.VMEM((CHUNKS, K), jnp.int32),
            pltpu.VMEM((K, DEGW), jnp.float32),
            pltpu.VMEM_SHARED((N_PAD, DEGW), jnp.float32),
        ],
    )
    def sc_degree(dst_hbm, ones_hbm, zeros1_hbm, out_hbm, dst_v, ones_v, accum):
        c = lax.axis_index("c")
        s = lax.axis_index("s")
        w = c * NS + s
        pltpu.sync_copy(zeros1_hbm, accum.at[pl.ds(s * RPT, RPT)])
        pltpu.sync_copy(dst_hbm.at[w], dst_v)
        pltpu.sync_copy(ones_hbm, ones_v)
        plsc.subcore_barrier()

        def body(i, carry):
            pltpu.sync_copy(ones_v, accum.at[dst_v.at[i]], add=True)
            return carry

        lax.fori_loop(0, CHUNKS, body, 0)
        plsc.subcore_barrier()
        pltpu.sync_copy(accum.at[pl.ds(s * RPT, RPT)],
                        out_hbm.at[c, pl.ds(s * RPT, RPT)])

    @functools.partial(
        pl.kernel,
        mesh=mesh,
        out_type=jax.ShapeDtypeStruct((NC, N_PAD, D), jnp.float32),
        scratch_types=[
            pltpu.VMEM((CHUNKS, K), jnp.int32),
            pltpu.VMEM((1, K), jnp.int32),
            pltpu.VMEM((1, K), jnp.int32),
            pltpu.VMEM((K, D), jnp.float32),
            pltpu.VMEM((K, D), jnp.float32),
            pltpu.VMEM_SHARED((N_PAD, D), jnp.float32),
            pltpu.SemaphoreType.DMA,
            pltpu.SemaphoreType.DMA,
            pltpu.SemaphoreType.DMA,
            pltpu.SemaphoreType.DMA,
            pltpu.SemaphoreType.DMA,
            pltpu.SemaphoreType.DMA,
            pltpu.SemaphoreType.DMA,
        ],
    )
    def sc_scatter(g_hbm, src_hbm, dst_hbm, zeros_hbm, out_hbm,
                   src_v, dbuf0, dbuf1, rows0, rows1, accum,
                   gsem0, gsem1, ssem0, ssem1, dsem0, dsem1, zsem):
        c = lax.axis_index("c")
        s = lax.axis_index("s")
        w = c * NS + s

        zc = pltpu.async_copy(zeros_hbm, accum.at[pl.ds(s * RPT, RPT)], zsem)
        sv = pltpu.async_copy(src_hbm.at[w], src_v, gsem1)
        pltpu.async_copy(dst_hbm.at[w, 0], dbuf0, dsem0)
        sv.wait()
        pltpu.async_copy(g_hbm.at[src_v.at[0]], rows0, gsem0)
        pltpu.async_copy(dst_hbm.at[w, 1], dbuf1, dsem1)
        zc.wait()
        plsc.subcore_barrier()

        def body(j, carry):
            e0 = 2 * j
            pltpu.make_async_copy(g_hbm.at[src_v.at[e0]], rows0, gsem0).wait()
            pltpu.make_async_copy(dst_hbm.at[w, e0], dbuf0, dsem0).wait()
            sc0 = pltpu.async_copy(rows0, accum.at[dbuf0.at[0]], ssem0,
                                   add=True)
            g1 = pltpu.async_copy(g_hbm.at[src_v.at[e0 + 1]], rows1, gsem1)
            sc0.wait()
            pltpu.async_copy(dst_hbm.at[w, e0 + 2], dbuf0, dsem0)
            g1.wait()
            pltpu.make_async_copy(dst_hbm.at[w, e0 + 1], dbuf1, dsem1).wait()
            sc1 = pltpu.async_copy(rows1, accum.at[dbuf1.at[0]], ssem1,
                                   add=True)
            pltpu.async_copy(g_hbm.at[src_v.at[e0 + 2]], rows0, gsem0)
            sc1.wait()
            e3 = jnp.minimum(e0 + 3, CHUNKS - 1)
            pltpu.async_copy(dst_hbm.at[w, e3], dbuf1, dsem1)
            return carry

        lax.fori_loop(0, (CHUNKS - 1) // 2, body, 0)
        pltpu.make_async_copy(
            g_hbm.at[src_v.at[CHUNKS - 1]], rows0, gsem0).wait()
        pltpu.make_async_copy(
            dst_hbm.at[w, CHUNKS - 1], dbuf0, dsem0).wait()
        pltpu.sync_copy(rows0, accum.at[dbuf0.at[0]], add=True)
        pltpu.make_async_copy(
            dst_hbm.at[w, CHUNKS - 1], dbuf1, dsem1).wait()
        plsc.subcore_barrier()
        pltpu.sync_copy(accum.at[pl.ds(s * RPT, RPT)],
                        out_hbm.at[c, pl.ds(s * RPT, RPT)])

    return sc_degree, sc_scatter


def _tc_norm_mm_body(x_ref, w_ref, deg_ref, g_ref, d_ref):
    deg = deg_ref[0][:, :1] + deg_ref[1][:, :1] + 1.0
    d = lax.rsqrt(deg)
    d_ref[...] = d
    g_ref[...] = jnp.dot(x_ref[...], w_ref[...],
                         preferred_element_type=jnp.float32) * d


def _tc_mid_body(s_ref, g1_ref, d_ref, b1_ref, w2_ref, g2_ref):
    d = d_ref[...]
    ssum = s_ref[0] + s_ref[1]
    out1 = jnp.maximum(d * (ssum + g1_ref[...]) + b1_ref[...], 0.0)
    g2_ref[...] = jnp.dot(out1, w2_ref[...],
                          preferred_element_type=jnp.float32) * d


def _tc_final_body(s_ref, g2_ref, d_ref, b2_ref, x_ref, o_ref):
    d = d_ref[...]
    ssum = s_ref[0] + s_ref[1]
    o_ref[...] = jnp.maximum(
        d * (ssum + g2_ref[...]) + b2_ref[...] + x_ref[...], 0.0)


_row_spec = pl.BlockSpec((BLK, D), lambda i: (i, 0))
_d_spec = pl.BlockSpec((BLK, 1), lambda i: (i, 0))
_part_spec = pl.BlockSpec((NC, BLK, D), lambda i: (0, i, 0))
_deg_spec = pl.BlockSpec((NC, BLK, DEGW), lambda i: (0, i, 0))
_w_spec = pl.BlockSpec((D, D), lambda i: (0, 0))
_b_spec = pl.BlockSpec((1, D), lambda i: (0, 0))

_tc_norm_mm = pl.pallas_call(
    _tc_norm_mm_body,
    grid=(GRID,),
    in_specs=[_row_spec, _w_spec, _deg_spec],
    out_specs=[_row_spec, _d_spec],
    out_shape=[
        jax.ShapeDtypeStruct((N_NODES, D), jnp.float32),
        jax.ShapeDtypeStruct((N_PAD, 1), jnp.float32),
    ],
)

_tc_mid = pl.pallas_call(
    _tc_mid_body,
    grid=(GRID,),
    in_specs=[_part_spec, _row_spec, _d_spec, _b_spec, _w_spec],
    out_specs=_row_spec,
    out_shape=jax.ShapeDtypeStruct((N_NODES, D), jnp.float32),
)

_tc_final = pl.pallas_call(
    _tc_final_body,
    grid=(GRID,),
    in_specs=[_part_spec, _row_spec, _d_spec, _b_spec, _row_spec],
    out_specs=_row_spec,
    out_shape=jax.ShapeDtypeStruct((N_NODES, D), jnp.float32),
)


@jax.jit
def kernel(x, edge_index, W1, b1, W2, b2):
    src = edge_index[0].astype(jnp.int32).reshape(NW, CHUNKS, K)
    dst = edge_index[1].astype(jnp.int32).reshape(NW, CHUNKS, K)
    dst4 = dst.reshape(NW, CHUNKS, 1, K)
    ones = jnp.ones((K, DEGW), jnp.float32)
    zeros = jnp.zeros((RPT, D), jnp.float32)
    zeros1 = zeros
    b1r = b1.reshape(1, D)
    b2r = b2.reshape(1, D)

    sc_degree, sc_scatter = _sc_kernels()
    deg_parts = sc_degree(dst, ones, zeros1)
    g1, d = _tc_norm_mm(x, W1, deg_parts)
    s1 = sc_scatter(g1, src, dst4, zeros)
    g2 = _tc_mid(s1, g1, d, b1r, W2)
    s2 = sc_scatter(g2, src, dst4, zeros)
    return _tc_final(s2, g2, d, b2r, x)
